# baseline (speedup 1.0000x reference)
import functools

import jax
import jax.numpy as jnp
from jax.experimental import pallas as pl


_TAU = 0.9
_P = 0.8
_MIN_K = 1
_E = 16


def _router_block(patch_ref, convw_ref, convb_ref, fcw_ref, fcb_ref, out_ref):
    x = patch_ref[...]
    w = convw_ref[...]
    y = jax.lax.dot_general(
        x, w, (((1,), (1,)), ((), ())),
        preferred_element_type=jnp.float32)
    y = jnp.maximum(y + convb_ref[...][None, :, :], 0.0)
    pooled = jnp.mean(y, axis=1)
    logits = jax.lax.dot_general(
        pooled, fcw_ref[...], (((1,), (1,)), ((), ())),
        preferred_element_type=jnp.float32) + fcb_ref[...]

    z = logits * (1.0 / _TAU)
    z = z - jnp.max(z, axis=-1, keepdims=True)
    e = jnp.exp(z)
    probs = e / jnp.sum(e, axis=-1, keepdims=True)

    v_i = probs[:, :, None]
    v_j = probs[:, None, :]
    idx = jax.lax.broadcasted_iota(jnp.int32, (_E, _E), 0)
    jdx = jax.lax.broadcasted_iota(jnp.int32, (_E, _E), 1)
    prec_incl = (v_j > v_i) | ((v_j == v_i) & (jdx <= idx)[None, :, :])
    cums = jnp.sum(jnp.where(prec_incl, jnp.broadcast_to(v_j, prec_incl.shape), 0.0),
                   axis=-1)
    rank = jnp.sum(prec_incl, axis=-1) - 1
    keep = (cums <= _P) | (rank < _MIN_K)
    masked = jnp.where(keep, probs, 0.0)
    denom = jnp.clip(jnp.sum(masked, axis=-1, keepdims=True), 1e-10, None)
    out_ref[...] = masked / denom


@functools.partial(jax.jit, static_argnames=())
def _run(patch, conv_w, conv_b, fc_w, fc_b):
    B = patch.shape[0]
    BC = 256
    x = patch.reshape(B, 196, 64)
    conv_b2 = conv_b.reshape(1, 128)
    fc_b2 = fc_b.reshape(1, _E)
    return pl.pallas_call(
        _router_block,
        grid=(B // BC,),
        in_specs=[
            pl.BlockSpec((BC, 196, 64), lambda i: (i, 0, 0)),
            pl.BlockSpec((128, 196), lambda i: (0, 0)),
            pl.BlockSpec((1, 128), lambda i: (0, 0)),
            pl.BlockSpec((_E, 128), lambda i: (0, 0)),
            pl.BlockSpec((1, _E), lambda i: (0, 0)),
        ],
        out_specs=pl.BlockSpec((BC, _E), lambda i: (i, 0)),
        out_shape=jax.ShapeDtypeStruct((B, _E), jnp.float32),
    )(x, conv_w, conv_b2, fc_w, fc_b2)


def kernel(patch, conv_w, conv_b, fc_w, fc_b, layer_idx, threshold):
    return _run(patch, conv_w, conv_b, fc_w, fc_b)
